# baseline (speedup 1.0000x reference)
import jax
import jax.numpy as jnp
from jax.experimental import pallas as pl
from jax.experimental.pallas import tpu as pltpu

_B = 4096
_T = 750
_WIN = 11
_SIGMA = 1.0
_E_THETA = 0.1
_E_G = 1.0
_E_ALPHA = 1.0
_TC = 128
_G = 6
_CB = 128
_AEH = 5376
_SW = 256
_NS_THRESH = 350.0
_JL = [j for j in range(_WIN) if j != 6]


def _loss_body(a_cur, b_cur, fb_ref, out_ref, g_ref, s4_ref, s3_ref, ae_ref):
    g = pl.program_id(0)

    lane = jax.lax.broadcasted_iota(jnp.int32, (1, _TC), 1) + g * _TC
    valid = lane < _T
    big = jnp.full((1, _TC), 1e9, jnp.float32)

    inv_two_sigma2 = jnp.float32(-0.5 / (_SIGMA * _SIGMA))
    dn = (((0,), (0,)), ((), ()))

    def fill_chunk(i, acc):
        r = pl.ds(i * _CB, _CB)
        s4_ref[r, 6:6 + _TC] = a_cur[r, :]
        s4_ref[r, 0:6] = fb_ref[0, r, 0:6]
        s4_ref[r, 6 + _TC:16 + _TC] = fb_ref[0, r, 6:16]
        d = a_cur[r, :] - b_cur[r, :]
        return acc + jnp.sum((d * d).reshape(_CB // 8, 8, _TC), axis=0)

    th = jax.lax.fori_loop(0, _B // _CB, fill_chunk,
                           jnp.zeros((8, _TC), jnp.float32))
    base = jnp.sum(th, axis=0, keepdims=True) * jnp.float32(_E_THETA)
    out_ref[...] = jnp.where(valid, base, 0.0).reshape(1, 1, _TC)

    @pl.when(g == _G - 1)
    def _():
        def patch(i, _):
            r = pl.ds(i * _CB, _CB)
            s4_ref[r, 116:121] = fb_ref[0, r, 32:37]
            return 0
        jax.lax.fori_loop(0, _B // _CB, patch, 0)

    s4a = s4_ref[:, :_TC]
    s4b = s4_ref[:, _TC:]
    g_aa = jax.lax.dot_general(s4a, s4a, dn, preferred_element_type=jnp.float32)
    g_ab = jax.lax.dot_general(s4a, s4b, dn, preferred_element_type=jnp.float32)
    g_bb = jax.lax.dot_general(s4b, s4b, dn, preferred_element_type=jnp.float32)
    g_ref[:_TC, :_TC] = g_aa
    g_ref[:_TC, _TC:] = g_ab
    g_ref[_TC:, :_TC] = g_ab.T
    g_ref[_TC:, _TC:] = g_bb

    rr = jax.lax.broadcasted_iota(jnp.int32, (_TC, _TC), 0)
    cc = jax.lax.broadcasted_iota(jnp.int32, (_TC, _TC), 1)
    eye = (rr == cc).astype(jnp.float32)

    def diag_at(row0, col0):
        blk = g_ref[row0:row0 + _TC, col0:col0 + _TC]
        return jnp.sum(blk * eye, axis=0, keepdims=True)

    cs_a = diag_at(0, 0)
    cs_b = diag_at(_TC, _TC)
    cs = jnp.concatenate([cs_a, cs_b], axis=1)
    cs6 = cs[:, 6:6 + _TC]
    min_ns = None
    for j in _JL:
        nsj = cs6 + cs[:, j:j + _TC] - 2.0 * diag_at(6, j)
        nsj = jnp.where(valid, nsj, big)
        min_ns = nsj if min_ns is None else jnp.minimum(min_ns, nsj)
    any_live = jnp.min(min_ns) < jnp.float32(_NS_THRESH)

    @pl.when(any_live)
    def _():
        def s3_chunk(i, _):
            r = pl.ds(i * _CB, _CB)
            s3_ref[r, 6:6 + _TC] = b_cur[r, :]
            s3_ref[r, 0:6] = fb_ref[0, r, 16:22]
            s3_ref[r, 6 + _TC:16 + _TC] = fb_ref[0, r, 22:32]

            @pl.when(g == _G - 1)
            def _():
                s3_ref[r, 116:121] = fb_ref[0, r, 37:42]

            return 0

        jax.lax.fori_loop(0, _B // _CB, s3_chunk, 0)

        def ns_chunk(i, carry):
            rows = pl.ds(i * _CB, _CB)
            ac = s4_ref[rows, 6:6 + _TC]
            new = [None] * len(_JL)
            for jj, j in enumerate(_JL):
                d = ac - s4_ref[rows, j:j + _TC]
                new[jj] = carry[jj] + jnp.sum(
                    (d * d).reshape(_CB // 8, 8, _TC), axis=0)
            return tuple(new)

        zeros = jnp.zeros((8, _TC), jnp.float32)
        ns_acc = jax.lax.fori_loop(0, _B // _CB, ns_chunk,
                                   (zeros,) * len(_JL))
        ens = [jnp.exp(inv_two_sigma2 *
                       jnp.where(valid,
                                 jnp.sum(ns_acc[jj], axis=0, keepdims=True),
                                 big))
               for jj in range(len(_JL))]

        def ae_fill(i, _):
            src = jax.lax.rem(i * _CB, jnp.int32(_B))
            ae_ref[pl.ds(i * _CB, _CB), :] = s4_ref[pl.ds(src, _CB),
                                                    6:6 + _TC]
            return 0

        jax.lax.fori_loop(0, _AEH // _CB, ae_fill, 0)

        def l1_chunk(i, tot):
            rows = pl.ds(i * _CB, _CB)
            s = jax.lax.rem(jnp.int32(11) * _CB * i, jnp.int32(_B))
            mw = ae_ref[pl.Slice(s, _CB, _WIN), :] - s4_ref[rows, 0:_TC]
            for j in range(1, _WIN):
                mw = jnp.maximum(
                    mw, ae_ref[pl.Slice(s + j, _CB, _WIN), :]
                    - s4_ref[rows, j:j + _TC])
            eg = jnp.exp(inv_two_sigma2 * jnp.float32(_E_G) * mw * mw)
            a2 = s3_ref[rows, 6:6 + _TC]
            acc = None
            for jj, j in enumerate(_JL):
                t = jnp.minimum(ens[jj], eg) * jnp.abs(
                    a2 - s3_ref[rows, j:j + _TC])
                acc = t if acc is None else acc + t
            return tot + jnp.sum(acc.reshape(_CB // 8, 8, _TC), axis=0)

        tot = jax.lax.fori_loop(0, _B // _CB, l1_chunk,
                                jnp.zeros((8, _TC), jnp.float32))
        part = jnp.sum(tot, axis=0, keepdims=True)
        out_ref[...] += jnp.where(valid, part, 0.0).reshape(1, 1, _TC)


def kernel(actioness, actioness_2):
    b = actioness.shape[0]
    front4 = jnp.tile(actioness[:, 0], 6).reshape(b, 6)
    back4 = jnp.tile(actioness[:, -1], 6).reshape(b, 6)[:, 1:]
    front3 = jnp.tile(actioness_2[:, 0], 6).reshape(b, 6)
    back3 = jnp.tile(actioness_2[:, -1], 6).reshape(b, 6)[:, 1:]
    z6 = jnp.zeros((b, 6), jnp.float32)
    z10 = jnp.zeros((b, 10), jnp.float32)
    steps = []
    for g in range(_G):
        hl4 = front4 if g == 0 else actioness[:, g * _TC - 6:g * _TC]
        hl3 = front3 if g == 0 else actioness_2[:, g * _TC - 6:g * _TC]
        hr4 = z10 if g == _G - 1 else actioness[:, g * _TC + _TC:
                                                g * _TC + _TC + 10]
        hr3 = z10 if g == _G - 1 else actioness_2[:, g * _TC + _TC:
                                                  g * _TC + _TC + 10]
        bk4 = back4 if g == _G - 1 else z6[:, :5]
        bk3 = back3 if g == _G - 1 else z6[:, :5]
        steps.append(jnp.concatenate(
            [hl4, hr4, hl3, hr3, bk4, bk3, z6], axis=1))
    fb = jnp.stack(steps, axis=0)

    cur = pl.BlockSpec((_B, _TC), lambda i: (0, i))
    fb_spec = pl.BlockSpec((1, _B, 48), lambda i: (i, 0, 0))

    partials = pl.pallas_call(
        _loss_body,
        grid=(_G,),
        in_specs=[cur, cur, fb_spec],
        out_specs=pl.BlockSpec((1, 1, _TC), lambda i: (i, 0, 0)),
        out_shape=jax.ShapeDtypeStruct((_G, 1, _TC), jnp.float32),
        scratch_shapes=[
            pltpu.VMEM((2 * _TC, 2 * _TC), jnp.float32),
            pltpu.VMEM((_B, _SW), jnp.float32),
            pltpu.VMEM((_B, _SW), jnp.float32),
            pltpu.VMEM((_AEH, _TC), jnp.float32),
        ],
        compiler_params=pltpu.CompilerParams(
            dimension_semantics=("parallel",),
            vmem_limit_bytes=48 * 1024 * 1024,
        ),
        name="act_eloss_v3",
    )(actioness, actioness_2, fb)

    return jnp.float32(_E_ALPHA / _B) * jnp.sum(partials)


# speedup vs baseline: 1.2109x; 1.2109x over previous
import jax
import jax.numpy as jnp
from jax.experimental import pallas as pl
from jax.experimental.pallas import tpu as pltpu

_B = 4096
_T = 750
_WIN = 11
_SIGMA = 1.0
_E_THETA = 0.1
_E_G = 1.0
_E_ALPHA = 1.0
_TC = 128
_G = 6
_PW = (_G + 1) * _TC
_CB = 128
_AEH = 5376
_NS_THRESH = 350.0


def _loss_body(p4a_ref, p4b_ref, p3a_ref, p3b_ref, out_ref, g_ref, ae_ref):
    g = pl.program_id(0)

    def win(aref, bref, r, j):
        rows = pl.ds(r, _CB)
        if j == 0:
            return aref[rows, :]
        return jnp.concatenate([aref[rows, j:], bref[rows, :j]], axis=1)

    def fold8(x):
        return jnp.sum(x.reshape(_CB // 8, 8, _TC), axis=0)

    lane = jax.lax.broadcasted_iota(jnp.int32, (1, _TC), 1) + g * _TC
    valid = lane < _T
    ns_bias = jnp.where(valid, 0.0, jnp.float32(1e9))

    inv_two_sigma2 = jnp.float32(-0.5 / (_SIGMA * _SIGMA))
    dn = (((0,), (0,)), ((), ()))

    a4 = p4a_ref[...]
    b4 = p4b_ref[...]
    g_aa = jax.lax.dot_general(a4, a4, dn, preferred_element_type=jnp.float32)
    g_ab = jax.lax.dot_general(a4, b4, dn, preferred_element_type=jnp.float32)
    g_bb = jax.lax.dot_general(b4, b4, dn, preferred_element_type=jnp.float32)
    g_ref[:_TC, :_TC] = g_aa
    g_ref[:_TC, _TC:] = g_ab
    g_ref[_TC:, :_TC] = g_ab.T
    g_ref[_TC:, _TC:] = g_bb

    rr = jax.lax.broadcasted_iota(jnp.int32, (_TC, _TC), 0)
    cc = jax.lax.broadcasted_iota(jnp.int32, (_TC, _TC), 1)
    eye = (rr == cc).astype(jnp.float32)

    def diag_at(row0, col0):
        blk = g_ref[row0:row0 + _TC, col0:col0 + _TC]
        return jnp.sum(blk * eye, axis=0, keepdims=True)

    cs_a = diag_at(0, 0)
    cs_b = diag_at(_TC, _TC)
    cs = jnp.concatenate([cs_a, cs_b], axis=1)
    cs6 = cs[:, 6:6 + _TC]
    min_ns = None
    for j in range(_WIN):
        if j == 6:
            continue
        nsj = cs6 + cs[:, j:j + _TC] - 2.0 * diag_at(6, j) + ns_bias
        min_ns = nsj if min_ns is None else jnp.minimum(min_ns, nsj)
    any_live = jnp.min(min_ns) < jnp.float32(_NS_THRESH)

    pcol = lane
    tvalid = (pcol >= 6) & (pcol < _T + 6)

    def th_chunk(i, acc):
        r = pl.ds(i * _CB, _CB)
        d = p4a_ref[r, :] - p3a_ref[r, :]
        return acc + fold8(d * d)

    th = jax.lax.fori_loop(0, _B // _CB, th_chunk,
                           jnp.zeros((8, _TC), jnp.float32))
    base = jnp.sum(th, axis=0, keepdims=True) * jnp.float32(_E_THETA)
    out_ref[...] = jnp.where(tvalid, base, 0.0).reshape(1, 1, _TC)

    @pl.when(any_live)
    def _():
        jlist = [j for j in range(_WIN) if j != 6]

        def ns_chunk(i, carry):
            r = i * _CB
            ac = win(p4a_ref, p4b_ref, r, 6)
            new = [None] * len(jlist)
            for jj, j in enumerate(jlist):
                d = ac - win(p4a_ref, p4b_ref, r, j)
                new[jj] = carry[jj] + fold8(d * d)
            return tuple(new)

        zeros = jnp.zeros((8, _TC), jnp.float32)
        ns_acc = jax.lax.fori_loop(0, _B // _CB, ns_chunk,
                                   (zeros,) * len(jlist))
        ens = [jnp.exp(inv_two_sigma2 *
                       (jnp.sum(ns_acc[jj], axis=0, keepdims=True) + ns_bias))
               for jj in range(len(jlist))]

        def ae_fill(i, _):
            src = jax.lax.rem(i * _CB, jnp.int32(_B))
            ae_ref[pl.ds(i * _CB, _CB), :] = win(p4a_ref, p4b_ref, src, 6)
            return 0

        jax.lax.fori_loop(0, _AEH // _CB, ae_fill, 0)

        def l1_chunk(i, tot):
            r = i * _CB
            s = jax.lax.rem(jnp.int32(11) * _CB * i, jnp.int32(_B))
            mw = (ae_ref[pl.Slice(s, _CB, _WIN), :]
                  - win(p4a_ref, p4b_ref, r, 0))
            for j in range(1, _WIN):
                mw = jnp.maximum(
                    mw, ae_ref[pl.Slice(s + j, _CB, _WIN), :]
                    - win(p4a_ref, p4b_ref, r, j))
            eg = jnp.exp(inv_two_sigma2 * jnp.float32(_E_G) * mw * mw)
            a2 = win(p3a_ref, p3b_ref, r, 6)
            acc = None
            for jj, j in enumerate(jlist):
                t = jnp.minimum(ens[jj], eg) * jnp.abs(
                    a2 - win(p3a_ref, p3b_ref, r, j))
                acc = t if acc is None else acc + t
            return tot + fold8(acc)

        tot = jax.lax.fori_loop(0, _B // _CB, l1_chunk,
                                jnp.zeros((8, _TC), jnp.float32))
        part = jnp.sum(tot, axis=0, keepdims=True)
        out_ref[...] += jnp.where(valid, part, 0.0).reshape(1, 1, _TC)


def _prep_body(ap_ref, ac_ref, bp_ref, bc_ref, fb_ref, o4_ref, o3_ref):
    k = pl.program_id(0)
    zeros7 = jnp.zeros((_CB, 7), jnp.float32)

    def chunk(i, _):
        r = pl.ds(i * _CB, _CB)
        o4_ref[r, :] = jnp.concatenate(
            [ap_ref[r, 122:], ac_ref[r, :122]], axis=1)
        o3_ref[r, :] = jnp.concatenate(
            [bp_ref[r, 122:], bc_ref[r, :122]], axis=1)

        @pl.when(k == 0)
        def _():
            o4_ref[r, 0:6] = fb_ref[r, 0:6]
            o3_ref[r, 0:6] = fb_ref[r, 11:17]

        @pl.when(k == 5)
        def _():
            o4_ref[r, 116:121] = fb_ref[r, 6:11]
            o3_ref[r, 116:121] = fb_ref[r, 17:22]
            o4_ref[r, 121:] = zeros7
            o3_ref[r, 121:] = zeros7

        @pl.when(k == 6)
        def _():
            o4_ref[r, :] = jnp.zeros((_CB, _TC), jnp.float32)
            o3_ref[r, :] = jnp.zeros((_CB, _TC), jnp.float32)

        return 0

    jax.lax.fori_loop(0, _B // _CB, chunk, 0)


def kernel(actioness, actioness_2):
    b = actioness.shape[0]
    fb = jnp.concatenate(
        [jnp.tile(actioness[:, 0], 6).reshape(b, 6),
         jnp.tile(actioness[:, -1], 6).reshape(b, 6)[:, 1:],
         jnp.tile(actioness_2[:, 0], 6).reshape(b, 6),
         jnp.tile(actioness_2[:, -1], 6).reshape(b, 6)[:, 1:],
         jnp.zeros((b, 10), jnp.float32)], axis=1)

    rprev = pl.BlockSpec(
        (_B, _TC), lambda k: (0, jnp.minimum(jnp.maximum(k - 1, 0), 5)))
    rcur = pl.BlockSpec((_B, _TC), lambda k: (0, jnp.minimum(k, 5)))
    fbs = pl.BlockSpec((_B, 32), lambda k: (0, 0))

    p4, p3 = pl.pallas_call(
        _prep_body,
        grid=(_G + 1,),
        in_specs=[rprev, rcur, rprev, rcur, fbs],
        out_specs=[pl.BlockSpec((_B, _TC), lambda k: (0, k)),
                   pl.BlockSpec((_B, _TC), lambda k: (0, k))],
        out_shape=[jax.ShapeDtypeStruct((_B, _PW), jnp.float32),
                   jax.ShapeDtypeStruct((_B, _PW), jnp.float32)],
        compiler_params=pltpu.CompilerParams(
            dimension_semantics=("parallel",),
            vmem_limit_bytes=48 * 1024 * 1024,
        ),
        name="act_eloss_prep",
    )(actioness, actioness, actioness_2, actioness_2, fb)

    col = pl.BlockSpec((_B, _TC), lambda i: (0, i))
    col_next = pl.BlockSpec((_B, _TC), lambda i: (0, i + 1))

    partials = pl.pallas_call(
        _loss_body,
        grid=(_G,),
        in_specs=[col, col_next, col, col_next],
        out_specs=pl.BlockSpec((1, 1, _TC), lambda i: (i, 0, 0)),
        out_shape=jax.ShapeDtypeStruct((_G, 1, _TC), jnp.float32),
        scratch_shapes=[
            pltpu.VMEM((2 * _TC, 2 * _TC), jnp.float32),
            pltpu.VMEM((_AEH, _TC), jnp.float32),
        ],
        compiler_params=pltpu.CompilerParams(
            dimension_semantics=("parallel",),
            vmem_limit_bytes=48 * 1024 * 1024,
        ),
        name="act_eloss_v3",
    )(p4, p4, p3, p3)

    return jnp.float32(_E_ALPHA / _B) * jnp.sum(partials)


# speedup vs baseline: 1.5360x; 1.2685x over previous
import jax
import jax.numpy as jnp
from jax.experimental import pallas as pl
from jax.experimental.pallas import tpu as pltpu

_B = 4096
_T = 750
_WIN = 11
_SIGMA = 1.0
_E_THETA = 0.1
_E_G = 1.0
_E_ALPHA = 1.0
_TC = 128
_G = 6
_PW = (_G + 1) * _TC
_CB = 128
_AEH = 5376
_NS_THRESH = 350.0


def _loss_body(p4a_ref, p4b_ref, p3a_ref, p3b_ref, out_ref, g_ref, ae_ref):
    g = pl.program_id(0)

    def win(aref, bref, r, j):
        rows = pl.ds(r, _CB)
        if j == 0:
            return aref[rows, :]
        return jnp.concatenate([aref[rows, j:], bref[rows, :j]], axis=1)

    def fold8(x):
        return jnp.sum(x.reshape(_CB // 8, 8, _TC), axis=0)

    lane = jax.lax.broadcasted_iota(jnp.int32, (1, _TC), 1) + g * _TC
    valid = lane < _T
    ns_bias = jnp.where(valid, 0.0, jnp.float32(1e9))

    inv_two_sigma2 = jnp.float32(-0.5 / (_SIGMA * _SIGMA))
    dn = (((0,), (0,)), ((), ()))

    a4 = p4a_ref[...]
    b4 = p4b_ref[...]
    g_aa = jax.lax.dot_general(a4, a4, dn, preferred_element_type=jnp.float32)
    g_ab = jax.lax.dot_general(a4, b4, dn, preferred_element_type=jnp.float32)
    g_bb = jax.lax.dot_general(b4, b4, dn, preferred_element_type=jnp.float32)
    g_ref[:_TC, :_TC] = g_aa
    g_ref[:_TC, _TC:] = g_ab
    g_ref[_TC:, :_TC] = g_ab.T
    g_ref[_TC:, _TC:] = g_bb

    rr = jax.lax.broadcasted_iota(jnp.int32, (_TC, _TC), 0)
    cc = jax.lax.broadcasted_iota(jnp.int32, (_TC, _TC), 1)
    eye = (rr == cc).astype(jnp.float32)

    def diag_at(row0, col0):
        blk = g_ref[row0:row0 + _TC, col0:col0 + _TC]
        return jnp.sum(blk * eye, axis=0, keepdims=True)

    cs_a = diag_at(0, 0)
    cs_b = diag_at(_TC, _TC)
    cs = jnp.concatenate([cs_a, cs_b], axis=1)
    cs6 = cs[:, 6:6 + _TC]
    min_ns = None
    for j in range(_WIN):
        if j == 6:
            continue
        nsj = cs6 + cs[:, j:j + _TC] - 2.0 * diag_at(6, j) + ns_bias
        min_ns = nsj if min_ns is None else jnp.minimum(min_ns, nsj)
    any_live = jnp.min(min_ns) < jnp.float32(_NS_THRESH)

    pcol = lane
    tvalid = (pcol >= 6) & (pcol < _T + 6)

    def th_chunk(i, acc):
        r = pl.ds(i * _CB, _CB)
        d = p4a_ref[r, :] - p3a_ref[r, :]
        return acc + fold8(d * d)

    th = jax.lax.fori_loop(0, _B // _CB, th_chunk,
                           jnp.zeros((8, _TC), jnp.float32))
    base = jnp.sum(th, axis=0, keepdims=True) * jnp.float32(_E_THETA)
    out_ref[...] = jnp.where(tvalid, base, 0.0).reshape(1, 1, _TC)

    @pl.when(any_live)
    def _():
        jlist = [j for j in range(_WIN) if j != 6]

        def ns_chunk(i, carry):
            r = i * _CB
            ac = win(p4a_ref, p4b_ref, r, 6)
            new = [None] * len(jlist)
            for jj, j in enumerate(jlist):
                d = ac - win(p4a_ref, p4b_ref, r, j)
                new[jj] = carry[jj] + fold8(d * d)
            return tuple(new)

        zeros = jnp.zeros((8, _TC), jnp.float32)
        ns_acc = jax.lax.fori_loop(0, _B // _CB, ns_chunk,
                                   (zeros,) * len(jlist))
        ens = [jnp.exp(inv_two_sigma2 *
                       (jnp.sum(ns_acc[jj], axis=0, keepdims=True) + ns_bias))
               for jj in range(len(jlist))]

        def ae_fill(i, _):
            src = jax.lax.rem(i * _CB, jnp.int32(_B))
            ae_ref[pl.ds(i * _CB, _CB), :] = win(p4a_ref, p4b_ref, src, 6)
            return 0

        jax.lax.fori_loop(0, _AEH // _CB, ae_fill, 0)

        def l1_chunk(i, tot):
            r = i * _CB
            s = jax.lax.rem(jnp.int32(11) * _CB * i, jnp.int32(_B))
            mw = (ae_ref[pl.Slice(s, _CB, _WIN), :]
                  - win(p4a_ref, p4b_ref, r, 0))
            for j in range(1, _WIN):
                mw = jnp.maximum(
                    mw, ae_ref[pl.Slice(s + j, _CB, _WIN), :]
                    - win(p4a_ref, p4b_ref, r, j))
            eg = jnp.exp(inv_two_sigma2 * jnp.float32(_E_G) * mw * mw)
            a2 = win(p3a_ref, p3b_ref, r, 6)
            acc = None
            for jj, j in enumerate(jlist):
                t = jnp.minimum(ens[jj], eg) * jnp.abs(
                    a2 - win(p3a_ref, p3b_ref, r, j))
                acc = t if acc is None else acc + t
            return tot + fold8(acc)

        tot = jax.lax.fori_loop(0, _B // _CB, l1_chunk,
                                jnp.zeros((8, _TC), jnp.float32))
        part = jnp.sum(tot, axis=0, keepdims=True)
        out_ref[...] += jnp.where(valid, part, 0.0).reshape(1, 1, _TC)


def _pad_like_ref(x):
    b = x.shape[0]
    front = jnp.tile(x[:, 0], 6).reshape(b, 6)
    back = jnp.tile(x[:, -1], 6).reshape(b, 6)
    zpad = jnp.zeros((b, _PW - (_T + _WIN)), x.dtype)
    return jnp.concatenate([front, x, back[:, 1:], zpad], axis=1)


def kernel(actioness, actioness_2):
    p4 = _pad_like_ref(actioness)
    p3 = _pad_like_ref(actioness_2)

    col = pl.BlockSpec((_B, _TC), lambda i: (0, i))
    col_next = pl.BlockSpec((_B, _TC), lambda i: (0, i + 1))

    partials = pl.pallas_call(
        _loss_body,
        grid=(_G,),
        in_specs=[col, col_next, col, col_next],
        out_specs=pl.BlockSpec((1, 1, _TC), lambda i: (i, 0, 0)),
        out_shape=jax.ShapeDtypeStruct((_G, 1, _TC), jnp.float32),
        scratch_shapes=[
            pltpu.VMEM((2 * _TC, 2 * _TC), jnp.float32),
            pltpu.VMEM((_AEH, _TC), jnp.float32),
        ],
        compiler_params=pltpu.CompilerParams(
            dimension_semantics=("parallel",),
            vmem_limit_bytes=48 * 1024 * 1024,
        ),
        name="act_eloss_v3",
    )(p4, p4, p3, p3)

    return jnp.float32(_E_ALPHA / _B) * jnp.sum(partials)
